# initial kernel scaffold (unmeasured)
import jax
import jax.numpy as jnp
from jax import lax
from jax.experimental import pallas as pl
from jax.experimental.pallas import tpu as pltpu

N_DEV = 4
HQ = 8
DH = 128
SQ = 2048
SKV = 2048
D_MODEL = 1024
BLK = 64
SCALE = 0.08838834764831843
CHUNK = SQ // N_DEV
N_STEPS = N_DEV - 1


def _body(x_ref, wq_ref, k_ref, v_ref, wo_ref, out_ref,
          ctx_ref, comm_ref, send_sems, recv_sems):
    my = lax.axis_index("i")
    left = (my + N_DEV - 1) % N_DEV
    right = (my + 1) % N_DEV

    barrier_sem = pltpu.get_barrier_semaphore()
    for nbr in (left, right):
        pl.semaphore_signal(
            barrier_sem, inc=1,
            device_id=(nbr,), device_id_type=pl.DeviceIdType.MESH,
        )
    pl.semaphore_wait(barrier_sem, 2)

    rows = lax.broadcasted_iota(jnp.int32, (SQ, SKV), 0) // BLK
    cols = lax.broadcasted_iota(jnp.int32, (SQ, SKV), 1) // BLK
    mask = (rows == cols) | (cols == 0) | (((rows + cols) % 3) == 0)

    for h in range(HQ):
        q_h = lax.dot_general(
            x_ref[...], wq_ref[:, h * DH:(h + 1) * DH],
            (((1,), (0,)), ((), ())), preferred_element_type=jnp.float32,
        )
        q_h = (q_h * SCALE).astype(jnp.bfloat16)
        s = lax.dot_general(
            q_h, k_ref[h],
            (((1,), (1,)), ((), ())), preferred_element_type=jnp.float32,
        )
        s = jnp.where(mask, s, -1e9)
        m = jnp.max(s, axis=1, keepdims=True)
        w = jnp.exp(s - m)
        w = (w / jnp.sum(w, axis=1, keepdims=True)).astype(jnp.bfloat16)
        ctx_h = lax.dot_general(
            w, v_ref[h],
            (((1,), (0,)), ((), ())), preferred_element_type=jnp.float32,
        )
        ctx_ref[:, h * DH:(h + 1) * DH] = ctx_h.astype(jnp.bfloat16)

    out_ref[...] = lax.dot_general(
        ctx_ref[...], wo_ref[...],
        (((1,), (0,)), ((), ())), preferred_element_type=jnp.float32,
    )

    for s_ in range(N_STEPS):
        send_c = (my - s_ + N_DEV) % N_DEV
        recv_c = (my - s_ - 1 + N_DEV) % N_DEV
        rdma = pltpu.make_async_remote_copy(
            src_ref=out_ref.at[pl.ds(send_c * CHUNK, CHUNK), :],
            dst_ref=comm_ref.at[s_],
            send_sem=send_sems.at[s_],
            recv_sem=recv_sems.at[s_],
            device_id=(right,),
            device_id_type=pl.DeviceIdType.MESH,
        )
        rdma.start()
        rdma.wait()
        out_ref[pl.ds(recv_c * CHUNK, CHUNK), :] = (
            out_ref[pl.ds(recv_c * CHUNK, CHUNK), :] + comm_ref[s_]
        )

    for s_ in range(N_STEPS):
        send_c = (my + 1 - s_ + N_DEV) % N_DEV
        recv_c = (my - s_ + N_DEV) % N_DEV
        slot = N_STEPS + s_
        rdma = pltpu.make_async_remote_copy(
            src_ref=out_ref.at[pl.ds(send_c * CHUNK, CHUNK), :],
            dst_ref=comm_ref.at[slot],
            send_sem=send_sems.at[slot],
            recv_sem=recv_sems.at[slot],
            device_id=(right,),
            device_id_type=pl.DeviceIdType.MESH,
        )
        rdma.start()
        rdma.wait()
        out_ref[pl.ds(recv_c * CHUNK, CHUNK), :] = comm_ref[slot]


def kernel(x, Wq, K_ext, V_ext, Wo):
    i = lax.axis_index("i")
    xb = x[0].astype(jnp.bfloat16)
    wq = Wq.astype(jnp.bfloat16)
    k = lax.dynamic_slice_in_dim(K_ext[0], i * HQ, HQ, axis=1)
    v = lax.dynamic_slice_in_dim(V_ext[0], i * HQ, HQ, axis=1)
    k = jnp.transpose(k, (1, 0, 2)).astype(jnp.bfloat16)
    v = jnp.transpose(v, (1, 0, 2)).astype(jnp.bfloat16)
    wo = Wo.astype(jnp.bfloat16)

    out = pl.pallas_call(
        _body,
        out_shape=jax.ShapeDtypeStruct((SQ, D_MODEL), jnp.float32),
        in_specs=[pl.BlockSpec(memory_space=pltpu.VMEM)] * 5,
        out_specs=pl.BlockSpec(memory_space=pltpu.VMEM),
        scratch_shapes=[
            pltpu.VMEM((SQ, D_MODEL), jnp.bfloat16),
            pltpu.VMEM((2 * N_STEPS, CHUNK, D_MODEL), jnp.float32),
            pltpu.SemaphoreType.DMA((2 * N_STEPS,)),
            pltpu.SemaphoreType.DMA((2 * N_STEPS,)),
        ],
        compiler_params=pltpu.CompilerParams(collective_id=0),
    )(xb, wq, k, v, wo)
    return out.reshape(1, SQ, D_MODEL)


# baseline (device time: 219237 ns/iter reference)
import jax
import jax.numpy as jnp
from jax import lax
from jax.experimental import pallas as pl
from jax.experimental.pallas import tpu as pltpu

N_DEV = 4
HQ = 8
DH = 128
SQ = 2048
SKV = 2048
D_MODEL = 1024
BLK = 64
SCALE = 0.08838834764831843
CHUNK = SQ // N_DEV
N_STEPS = N_DEV - 1
QT = 512
N_QT = SQ // QT


def _attn_tile(x_ref, wq_ref, k_ref, v_ref, wo_ref, out_ref, ctx_ref, t):
    r0 = t * QT
    rows = (lax.broadcasted_iota(jnp.int32, (QT, SKV), 0) + r0) // BLK
    cols = lax.broadcasted_iota(jnp.int32, (QT, SKV), 1) // BLK
    mask = (rows == cols) | (cols == 0) | (((rows + cols) % 3) == 0)
    x_t = x_ref[r0:r0 + QT, :]
    for h in range(HQ):
        q = lax.dot_general(
            x_t, wq_ref[:, h * DH:(h + 1) * DH],
            (((1,), (0,)), ((), ())), preferred_element_type=jnp.float32,
        )
        q = (q * SCALE).astype(jnp.bfloat16)
        s = lax.dot_general(
            q, k_ref[h],
            (((1,), (1,)), ((), ())), preferred_element_type=jnp.float32,
        )
        s = jnp.where(mask, s, -1e9)
        m = jnp.max(s, axis=1, keepdims=True)
        w = jnp.exp(s - m)
        w = (w / jnp.sum(w, axis=1, keepdims=True)).astype(jnp.bfloat16)
        ctx_h = lax.dot_general(
            w, v_ref[h],
            (((1,), (0,)), ((), ())), preferred_element_type=jnp.float32,
        )
        ctx_ref[:, h * DH:(h + 1) * DH] = ctx_h.astype(jnp.bfloat16)
    out_ref[r0:r0 + QT, :] = lax.dot_general(
        ctx_ref[...], wo_ref[...],
        (((1,), (0,)), ((), ())), preferred_element_type=jnp.float32,
    )


def _body(x_ref, wq_ref, k_ref, v_ref, wo_ref, out_ref,
          ctx_ref, stage_ref, comm_ref, send_sems, recv_sems):
    my = lax.axis_index("i")
    left = (my + N_DEV - 1) % N_DEV
    right = (my + 1) % N_DEV

    barrier_sem = pltpu.get_barrier_semaphore()
    for nbr in (left, right):
        pl.semaphore_signal(
            barrier_sem, inc=1,
            device_id=(nbr,), device_id_type=pl.DeviceIdType.MESH,
        )
    pl.semaphore_wait(barrier_sem, 2)

    for t in range(N_QT):
        _attn_tile(x_ref, wq_ref, k_ref, v_ref, wo_ref, out_ref, ctx_ref, t)

    for s_ in range(N_STEPS):
        send_c = (my - s_ + N_DEV) % N_DEV
        recv_c = (my - s_ - 1 + N_DEV) % N_DEV
        stage_ref[...] = out_ref[pl.ds(send_c * CHUNK, CHUNK), :].astype(
            jnp.bfloat16)
        rdma = pltpu.make_async_remote_copy(
            src_ref=stage_ref,
            dst_ref=comm_ref.at[s_],
            send_sem=send_sems.at[s_],
            recv_sem=recv_sems.at[s_],
            device_id=(right,),
            device_id_type=pl.DeviceIdType.MESH,
        )
        rdma.start()
        rdma.wait()
        out_ref[pl.ds(recv_c * CHUNK, CHUNK), :] = (
            out_ref[pl.ds(recv_c * CHUNK, CHUNK), :]
            + comm_ref[s_].astype(jnp.float32)
        )

    for s_ in range(N_STEPS):
        send_c = (my + 1 - s_ + N_DEV) % N_DEV
        recv_c = (my - s_ + N_DEV) % N_DEV
        slot = N_STEPS + s_
        if s_ == 0:
            stage_ref[...] = out_ref[pl.ds(send_c * CHUNK, CHUNK), :].astype(
                jnp.bfloat16)
            src = stage_ref
        else:
            src = comm_ref.at[slot - 1]
        rdma = pltpu.make_async_remote_copy(
            src_ref=src,
            dst_ref=comm_ref.at[slot],
            send_sem=send_sems.at[slot],
            recv_sem=recv_sems.at[slot],
            device_id=(right,),
            device_id_type=pl.DeviceIdType.MESH,
        )
        rdma.start()
        rdma.wait()
        out_ref[pl.ds(recv_c * CHUNK, CHUNK), :] = comm_ref[slot].astype(
            jnp.float32)


def kernel(x, Wq, K_ext, V_ext, Wo):
    i = lax.axis_index("i")
    xb = x[0].astype(jnp.bfloat16)
    wq = Wq.astype(jnp.bfloat16)
    k = lax.dynamic_slice_in_dim(K_ext[0], i * HQ, HQ, axis=1)
    v = lax.dynamic_slice_in_dim(V_ext[0], i * HQ, HQ, axis=1)
    k = jnp.transpose(k, (1, 0, 2)).astype(jnp.bfloat16)
    v = jnp.transpose(v, (1, 0, 2)).astype(jnp.bfloat16)
    wo = Wo.astype(jnp.bfloat16)

    out = pl.pallas_call(
        _body,
        out_shape=jax.ShapeDtypeStruct((SQ, D_MODEL), jnp.float32),
        in_specs=[pl.BlockSpec(memory_space=pltpu.VMEM)] * 5,
        out_specs=pl.BlockSpec(memory_space=pltpu.VMEM),
        scratch_shapes=[
            pltpu.VMEM((QT, D_MODEL), jnp.bfloat16),
            pltpu.VMEM((CHUNK, D_MODEL), jnp.bfloat16),
            pltpu.VMEM((2 * N_STEPS, CHUNK, D_MODEL), jnp.bfloat16),
            pltpu.SemaphoreType.DMA((2 * N_STEPS,)),
            pltpu.SemaphoreType.DMA((2 * N_STEPS,)),
        ],
        compiler_params=pltpu.CompilerParams(
            collective_id=0,
            vmem_limit_bytes=100 * 1024 * 1024,
        ),
    )(xb, wq, k, v, wo)
    return out.reshape(1, SQ, D_MODEL)
